# baseline (speedup 1.0000x reference)
import numpy as np
import jax
import jax.numpy as jnp
from jax.experimental import pallas as pl
from jax.experimental.pallas import tpu as pltpu

_RADIUS = 16
_TAU = 2.0
_T = 128
_NUM_SHIFTS = _RADIUS + 1
_K = 2 * _RADIUS + 1


def _build_shift_filter_matrix():
    a = 1.0 - 1.0 / _TAU
    b = 1.0 / _TAU
    t = np.arange(_T)
    diff = t[:, None] - t[None, :]
    A = np.where(diff >= 0, b * np.power(a, np.maximum(diff, 0)), 0.0)
    mats = []
    for s in range(_NUM_SHIFTS):
        cols = (np.arange(_T) + s) % _T
        mats.append(A[:, cols])
    return np.concatenate(mats, axis=0).astype(np.float32)


_BALL = _build_shift_filter_matrix()
_BASE0 = np.maximum(np.arange(_K) - _RADIUS, 0)
_BASE1 = np.maximum(_RADIUS - np.arange(_K), 0)

_RB = 128


def _build_deinterleave():
    d = np.zeros((2, 2 * _RB, _RB), np.float32)
    for r in range(_RB):
        d[0, 2 * r, r] = 1.0
        d[1, 2 * r + 1, r] = 1.0
    return d


_DEINT = _build_deinterleave()


def _body(x_ref, ball_ref, dint_ref, out_ref):
    ball = ball_ref[...]
    xmix = x_ref[...]
    fs, mm, fm = [], [], []
    for i in range(2):
        X = jnp.dot(xmix, dint_ref[i], preferred_element_type=jnp.float32)
        fs_i = jnp.dot(ball, X, preferred_element_type=jnp.float32)
        mx = jnp.max(X, axis=0, keepdims=True)
        ti = jax.lax.broadcasted_iota(jnp.int32, X.shape, 0)
        am = jnp.min(jnp.where(X == mx, ti, _T), axis=0, keepdims=True)
        mm_i = jnp.minimum(_T - 1 - am, _RADIUS)
        acc = fs_i[0:_T, :]
        for s in range(1, _NUM_SHIFTS):
            acc = jnp.where(mm_i == s, fs_i[s * _T:(s + 1) * _T, :], acc)
        fs.append(fs_i)
        mm.append(mm_i)
        fm.append(acc)
    for k in range(_K):
        b0 = int(_BASE0[k])
        b1 = int(_BASE1[k])
        p0 = jnp.where(b0 <= mm[0], fs[0][b0 * _T:(b0 + 1) * _T, :], fm[0])
        p1 = jnp.where(b1 <= mm[1], fs[1][b1 * _T:(b1 + 1) * _T, :], fm[1])
        out_ref[k] = p0 + p1


def kernel(input, _delay, weight):
    T, N, C, DI = input.shape
    NC = N * C
    G = NC // _RB
    xin = input.reshape(T, NC * DI)
    ball_w = jnp.asarray(_BALL) * weight.astype(jnp.float32)
    dint = jnp.asarray(_DEINT)
    out = pl.pallas_call(
        _body,
        grid=(G,),
        in_specs=[
            pl.BlockSpec((T, DI * _RB), lambda g: (0, g)),
            pl.BlockSpec((_NUM_SHIFTS * _T, _T), lambda g: (0, 0)),
            pl.BlockSpec((2, DI * _RB, _RB), lambda g: (0, 0, 0)),
        ],
        out_specs=pl.BlockSpec((_K, T, _RB), lambda g: (0, 0, g)),
        out_shape=jax.ShapeDtypeStruct((_K, T, NC), jnp.float32),
        compiler_params=pltpu.CompilerParams(
            dimension_semantics=("parallel",)),
    )(xin, ball_w, dint)
    eye = jnp.eye(_K, dtype=jnp.float32)
    y = jnp.einsum('ktr,kK->trK', out, eye)
    return y.reshape(T, N, C, _K)


# speedup vs baseline: 1.0676x; 1.0676x over previous
import numpy as np
import jax
import jax.numpy as jnp
from jax.experimental import pallas as pl
from jax.experimental.pallas import tpu as pltpu

_RADIUS = 16
_TAU = 2.0
_T = 128
_NUM_SHIFTS = _RADIUS + 1
_K = 2 * _RADIUS + 1
_RB = 128
_CHUNKS = 4


def _build_shift_filter_matrix():
    a = 1.0 - 1.0 / _TAU
    b = 1.0 / _TAU
    t = np.arange(_T)
    diff = t[:, None] - t[None, :]
    A = np.where(diff >= 0, b * np.power(a, np.maximum(diff, 0)), 0.0)
    mats = []
    for s in range(_NUM_SHIFTS):
        cols = (np.arange(_T) + s) % _T
        mats.append(A[:, cols])
    return np.concatenate(mats, axis=0).astype(np.float32)


_BALL = _build_shift_filter_matrix()
_BASE0 = np.maximum(np.arange(_K) - _RADIUS, 0)
_BASE1 = np.maximum(_RADIUS - np.arange(_K), 0)


def _body(x_ref, ball_ref, out_ref):
    ball = ball_ref[...]
    fs, mm, fm = [], [], []
    for i in range(2):
        X = x_ref[i]
        fs_i = jnp.dot(ball, X.astype(jnp.bfloat16),
                       preferred_element_type=jnp.float32)
        mx = jnp.max(X, axis=0, keepdims=True)
        ti = jax.lax.broadcasted_iota(jnp.int32, X.shape, 0)
        am = jnp.min(jnp.where(X == mx, ti, _T), axis=0, keepdims=True)
        mm_i = jnp.minimum(_T - 1 - am, _RADIUS)
        acc = fs_i[0:_T, :]
        for s in range(1, _NUM_SHIFTS):
            acc = jnp.where(mm_i == s, fs_i[s * _T:(s + 1) * _T, :], acc)
        fs.append(fs_i)
        mm.append(mm_i)
        fm.append(acc)
    for k in range(_K):
        b0 = int(_BASE0[k])
        b1 = int(_BASE1[k])
        p0 = jnp.where(b0 <= mm[0], fs[0][b0 * _T:(b0 + 1) * _T, :], fm[0])
        p1 = jnp.where(b1 <= mm[1], fs[1][b1 * _T:(b1 + 1) * _T, :], fm[1])
        out_ref[k] = p0 + p1


def _run_chunk(xr_chunk, ball_w):
    nc = xr_chunk.shape[2]
    out = pl.pallas_call(
        _body,
        grid=(nc // _RB,),
        in_specs=[
            pl.BlockSpec((2, _T, _RB), lambda g: (0, 0, g)),
            pl.BlockSpec((_NUM_SHIFTS * _T, _T), lambda g: (0, 0)),
        ],
        out_specs=pl.BlockSpec((_K, _T, _RB), lambda g: (0, 0, g)),
        out_shape=jax.ShapeDtypeStruct((_K, _T, nc), jnp.float32),
        compiler_params=pltpu.CompilerParams(
            dimension_semantics=("parallel",)),
    )(xr_chunk, ball_w)
    return jnp.transpose(out, (1, 2, 0))


def kernel(input, _delay, weight):
    T, N, C, DI = input.shape
    NC = N * C
    xr = jnp.transpose(input, (3, 0, 1, 2)).reshape(DI, T, NC)
    ball_w = (jnp.asarray(_BALL) *
              weight.astype(jnp.float32)).astype(jnp.bfloat16)
    step = NC // _CHUNKS
    parts = [_run_chunk(xr[:, :, j * step:(j + 1) * step], ball_w)
             for j in range(_CHUNKS)]
    y = jnp.concatenate(parts, axis=1)
    return y.reshape(T, N, C, _K)


# speedup vs baseline: 1.1919x; 1.1165x over previous
import numpy as np
import jax
import jax.numpy as jnp
from jax.experimental import pallas as pl
from jax.experimental.pallas import tpu as pltpu

_RADIUS = 16
_TAU = 2.0
_T = 128
_NUM_SHIFTS = _RADIUS + 1
_K = 2 * _RADIUS + 1
_RB = 128
_CHUNKS = 2


def _build_shift_filter_matrix():
    a = 1.0 - 1.0 / _TAU
    b = 1.0 / _TAU
    t = np.arange(_T)
    diff = t[:, None] - t[None, :]
    A = np.where(diff >= 0, b * np.power(a, np.maximum(diff, 0)), 0.0)
    mats = []
    for s in range(_NUM_SHIFTS):
        cols = (np.arange(_T) + s) % _T
        mats.append(A[:, cols])
    return np.concatenate(mats, axis=0).astype(np.float32)


_BALL = _build_shift_filter_matrix()
_BASE0 = np.maximum(np.arange(_K) - _RADIUS, 0)
_BASE1 = np.maximum(_RADIUS - np.arange(_K), 0)


def _body(x_ref, ball_ref, out_ref):
    ball = ball_ref[...]
    fs, mm, fm = [], [], []
    for i in range(2):
        X = x_ref[i]
        fs_i = jnp.dot(ball, X.astype(jnp.bfloat16),
                       preferred_element_type=jnp.float32)
        mx = jnp.max(X, axis=0, keepdims=True)
        ti = jax.lax.broadcasted_iota(jnp.int32, X.shape, 0)
        am = jnp.min(jnp.where(X == mx, ti, _T), axis=0, keepdims=True)
        mm_i = jnp.minimum(_T - 1 - am, _RADIUS)
        acc = fs_i[0:_T, :]
        for s in range(1, _NUM_SHIFTS):
            acc = jnp.where(mm_i == s, fs_i[s * _T:(s + 1) * _T, :], acc)
        fs.append(fs_i)
        mm.append(mm_i)
        fm.append(acc)
    for k in range(_K):
        b0 = int(_BASE0[k])
        b1 = int(_BASE1[k])
        p0 = jnp.where(b0 <= mm[0], fs[0][b0 * _T:(b0 + 1) * _T, :], fm[0])
        p1 = jnp.where(b1 <= mm[1], fs[1][b1 * _T:(b1 + 1) * _T, :], fm[1])
        out_ref[k] = (p0 + p1).astype(jnp.bfloat16)


def _run_chunk(xr_chunk, ball_w):
    nc = xr_chunk.shape[2]
    out = pl.pallas_call(
        _body,
        grid=(nc // _RB,),
        in_specs=[
            pl.BlockSpec((2, _T, _RB), lambda g: (0, 0, g)),
            pl.BlockSpec((_NUM_SHIFTS * _T, _T), lambda g: (0, 0)),
        ],
        out_specs=pl.BlockSpec((_K, _T, _RB), lambda g: (0, 0, g)),
        out_shape=jax.ShapeDtypeStruct((_K, _T, nc), jnp.bfloat16),
        compiler_params=pltpu.CompilerParams(
            dimension_semantics=("parallel",)),
    )(xr_chunk, ball_w)
    return jnp.transpose(out, (1, 2, 0)).astype(jnp.float32)


def kernel(input, _delay, weight):
    T, N, C, DI = input.shape
    NC = N * C
    xr = jnp.transpose(input, (3, 0, 1, 2)).reshape(DI, T, NC)
    ball_w = (jnp.asarray(_BALL) *
              weight.astype(jnp.float32)).astype(jnp.bfloat16)
    step = NC // _CHUNKS
    parts = [_run_chunk(xr[:, :, j * step:(j + 1) * step], ball_w)
             for j in range(_CHUNKS)]
    y = jnp.concatenate(parts, axis=1)
    return y.reshape(T, N, C, _K)


# speedup vs baseline: 1.2172x; 1.0212x over previous
import numpy as np
import jax
import jax.numpy as jnp
from jax.experimental import pallas as pl
from jax.experimental.pallas import tpu as pltpu

_RADIUS = 16
_TAU = 2.0
_T = 128
_NUM_SHIFTS = _RADIUS + 1
_K = 2 * _RADIUS + 1
_RB = 128
_CHUNKS = 1


def _build_shift_filter_matrix():
    a = 1.0 - 1.0 / _TAU
    b = 1.0 / _TAU
    t = np.arange(_T)
    diff = t[:, None] - t[None, :]
    A = np.where(diff >= 0, b * np.power(a, np.maximum(diff, 0)), 0.0)
    mats = []
    for s in range(_NUM_SHIFTS):
        cols = (np.arange(_T) + s) % _T
        mats.append(A[:, cols])
    return np.concatenate(mats, axis=0).astype(np.float32)


_BALL = _build_shift_filter_matrix()
_BASE0 = np.maximum(np.arange(_K) - _RADIUS, 0)
_BASE1 = np.maximum(_RADIUS - np.arange(_K), 0)


def _body(x_ref, ball_ref, out_ref):
    ball = ball_ref[...]
    fs, mm, fm = [], [], []
    for i in range(2):
        X = x_ref[i]
        fs_i = jnp.dot(ball, X.astype(jnp.bfloat16),
                       preferred_element_type=jnp.float32)
        mx = jnp.max(X, axis=0, keepdims=True)
        ti = jax.lax.broadcasted_iota(jnp.int32, X.shape, 0)
        am = jnp.min(jnp.where(X == mx, ti, _T), axis=0, keepdims=True)
        mm_i = jnp.minimum(_T - 1 - am, _RADIUS)
        acc = fs_i[0:_T, :]
        for s in range(1, _NUM_SHIFTS):
            acc = jnp.where(mm_i == s, fs_i[s * _T:(s + 1) * _T, :], acc)
        fs.append(fs_i)
        mm.append(mm_i)
        fm.append(acc)
    for k in range(_K):
        b0 = int(_BASE0[k])
        b1 = int(_BASE1[k])
        p0 = jnp.where(b0 <= mm[0], fs[0][b0 * _T:(b0 + 1) * _T, :], fm[0])
        p1 = jnp.where(b1 <= mm[1], fs[1][b1 * _T:(b1 + 1) * _T, :], fm[1])
        out_ref[k] = (p0 + p1).astype(jnp.bfloat16)


def _run_chunk(xr_chunk, ball_w):
    nc = xr_chunk.shape[2]
    out = pl.pallas_call(
        _body,
        grid=(nc // _RB,),
        in_specs=[
            pl.BlockSpec((2, _T, _RB), lambda g: (0, 0, g)),
            pl.BlockSpec((_NUM_SHIFTS * _T, _T), lambda g: (0, 0)),
        ],
        out_specs=pl.BlockSpec((_K, _T, _RB), lambda g: (0, 0, g)),
        out_shape=jax.ShapeDtypeStruct((_K, _T, nc), jnp.bfloat16),
        compiler_params=pltpu.CompilerParams(
            dimension_semantics=("parallel",)),
    )(xr_chunk, ball_w)
    return jnp.transpose(out, (1, 2, 0)).astype(jnp.float32)


def kernel(input, _delay, weight):
    T, N, C, DI = input.shape
    NC = N * C
    xr = jnp.transpose(input, (3, 0, 1, 2)).reshape(DI, T, NC)
    ball_w = (jnp.asarray(_BALL) *
              weight.astype(jnp.float32)).astype(jnp.bfloat16)
    step = NC // _CHUNKS
    parts = [_run_chunk(xr[:, :, j * step:(j + 1) * step], ball_w)
             for j in range(_CHUNKS)]
    y = jnp.concatenate(parts, axis=1)
    return y.reshape(T, N, C, _K)


# speedup vs baseline: 1.3987x; 1.1491x over previous
import numpy as np
import jax
import jax.numpy as jnp
from jax.experimental import pallas as pl
from jax.experimental.pallas import tpu as pltpu

_RADIUS = 16
_TAU = 2.0
_T = 128
_NUM_SHIFTS = _RADIUS + 1
_K = 2 * _RADIUS + 1
_RB = 128


def _build_shift_filter_matrix():
    a = 1.0 - 1.0 / _TAU
    b = 1.0 / _TAU
    t = np.arange(_T)
    diff = t[:, None] - t[None, :]
    A = np.where(diff >= 0, b * np.power(a, np.maximum(diff, 0)), 0.0)
    mats = []
    for s in range(_NUM_SHIFTS):
        cols = (np.arange(_T) + s) % _T
        mats.append(A[:, cols])
    return np.concatenate(mats, axis=0).astype(np.float32)


_BALL = _build_shift_filter_matrix()
_BASE0 = np.maximum(np.arange(_K) - _RADIUS, 0)
_BASE1 = np.maximum(_RADIUS - np.arange(_K), 0)


def _build_deinterleave():
    d = np.zeros((2, 2 * _RB, _RB), np.float32)
    for r in range(_RB):
        d[0, 2 * r, r] = 1.0
        d[1, 2 * r + 1, r] = 1.0
    return d


_DEINT = _build_deinterleave()


def _body(x_ref, ball_ref, dint_ref, out_ref):
    ball = ball_ref[...]
    xmix = x_ref[...]
    mxm = jnp.max(xmix, axis=0, keepdims=True)
    tim = jax.lax.broadcasted_iota(jnp.int32, xmix.shape, 0)
    am = jnp.min(jnp.where(xmix == mxm, tim, _T), axis=0, keepdims=True)
    mm_mix = jnp.minimum(_T - 1 - am, _RADIUS).astype(jnp.float32)
    xmix_b = xmix.astype(jnp.bfloat16)
    fs, mm, fm = [], [], []
    for i in range(2):
        mm_i = jnp.dot(mm_mix.astype(jnp.bfloat16), dint_ref[i],
                       preferred_element_type=jnp.float32)
        X = jnp.dot(xmix_b, dint_ref[i],
                    preferred_element_type=jnp.float32)
        fs_i = jnp.dot(ball, X.astype(jnp.bfloat16),
                       preferred_element_type=jnp.float32)
        acc = fs_i[0:_T, :]
        for s in range(1, _NUM_SHIFTS):
            acc = jnp.where(mm_i == s, fs_i[s * _T:(s + 1) * _T, :], acc)
        fs.append(fs_i)
        mm.append(mm_i)
        fm.append(acc)
    for k in range(_K):
        b0 = float(_BASE0[k])
        b1 = float(_BASE1[k])
        p0 = jnp.where(b0 <= mm[0], fs[0][int(_BASE0[k]) * _T:
                                          (int(_BASE0[k]) + 1) * _T, :], fm[0])
        p1 = jnp.where(b1 <= mm[1], fs[1][int(_BASE1[k]) * _T:
                                          (int(_BASE1[k]) + 1) * _T, :], fm[1])
        out_ref[k] = p0 + p1


def kernel(input, _delay, weight):
    T, N, C, DI = input.shape
    NC = N * C
    G = NC // _RB
    xin = input.reshape(T, NC * DI)
    ball_w = (jnp.asarray(_BALL) *
              weight.astype(jnp.float32)).astype(jnp.bfloat16)
    dint = jnp.asarray(_DEINT).astype(jnp.bfloat16)
    out = pl.pallas_call(
        _body,
        grid=(G,),
        in_specs=[
            pl.BlockSpec((T, DI * _RB), lambda g: (0, g)),
            pl.BlockSpec((_NUM_SHIFTS * _T, _T), lambda g: (0, 0)),
            pl.BlockSpec((2, DI * _RB, _RB), lambda g: (0, 0, 0)),
        ],
        out_specs=pl.BlockSpec((_K, T, _RB), lambda g: (0, 0, g)),
        out_shape=jax.ShapeDtypeStruct((_K, T, NC), jnp.float32),
        compiler_params=pltpu.CompilerParams(
            dimension_semantics=("parallel",)),
    )(xin, ball_w, dint)
    return jnp.transpose(out, (1, 2, 0)).reshape(T, N, C, _K)


# speedup vs baseline: 1.4157x; 1.0122x over previous
import numpy as np
import jax
import jax.numpy as jnp
from jax.experimental import pallas as pl
from jax.experimental.pallas import tpu as pltpu

_RADIUS = 16
_TAU = 2.0
_T = 128
_NUM_SHIFTS = _RADIUS + 1
_K = 2 * _RADIUS + 1
_RB = 128


def _build_shift_filter_matrix():
    a = 1.0 - 1.0 / _TAU
    b = 1.0 / _TAU
    t = np.arange(_T)
    diff = t[:, None] - t[None, :]
    A = np.where(diff >= 0, b * np.power(a, np.maximum(diff, 0)), 0.0)
    mats = []
    for s in range(_NUM_SHIFTS):
        cols = (np.arange(_T) + s) % _T
        mats.append(A[:, cols])
    return np.concatenate(mats, axis=0).astype(np.float32)


_BALL = _build_shift_filter_matrix()
_BASE0 = np.maximum(np.arange(_K) - _RADIUS, 0)
_BASE1 = np.maximum(_RADIUS - np.arange(_K), 0)


def _body(x_ref, ball_ref, out_ref):
    ball = ball_ref[...]
    fs, mm, fm = [], [], []
    for i in range(2):
        X = x_ref[i]
        fs_i = jnp.dot(ball, X.astype(jnp.bfloat16),
                       preferred_element_type=jnp.float32)
        mx = jnp.max(X, axis=0, keepdims=True)
        ti = jax.lax.broadcasted_iota(jnp.int32, X.shape, 0)
        am = jnp.min(jnp.where(X == mx, ti, _T), axis=0, keepdims=True)
        mm_i = jnp.minimum(_T - 1 - am, _RADIUS)
        acc = fs_i[0:_T, :]
        for s in range(1, _NUM_SHIFTS):
            acc = jnp.where(mm_i == s, fs_i[s * _T:(s + 1) * _T, :], acc)
        fs.append(fs_i)
        mm.append(mm_i)
        fm.append(acc)
    for k in range(_K):
        b0 = int(_BASE0[k])
        b1 = int(_BASE1[k])
        p0 = jnp.where(b0 <= mm[0], fs[0][b0 * _T:(b0 + 1) * _T, :], fm[0])
        p1 = jnp.where(b1 <= mm[1], fs[1][b1 * _T:(b1 + 1) * _T, :], fm[1])
        out_ref[k] = p0 + p1


def kernel(input, _delay, weight):
    T, N, C, DI = input.shape
    NC = N * C
    xr = jnp.transpose(input, (3, 0, 1, 2)).reshape(DI, T, NC)
    ball_w = (jnp.asarray(_BALL) *
              weight.astype(jnp.float32)).astype(jnp.bfloat16)
    out = pl.pallas_call(
        _body,
        grid=(NC // _RB,),
        in_specs=[
            pl.BlockSpec((DI, T, _RB), lambda g: (0, 0, g)),
            pl.BlockSpec((_NUM_SHIFTS * _T, _T), lambda g: (0, 0)),
        ],
        out_specs=pl.BlockSpec((_K, T, _RB), lambda g: (0, 0, g)),
        out_shape=jax.ShapeDtypeStruct((_K, T, NC), jnp.float32),
        compiler_params=pltpu.CompilerParams(
            dimension_semantics=("parallel",)),
    )(xr, ball_w)
    return jnp.transpose(out, (1, 2, 0)).reshape(T, N, C, _K)
